# baseline (speedup 1.0000x reference)
import functools

import jax
import jax.numpy as jnp
from jax import lax
from jax.experimental import pallas as pl
from jax.experimental.pallas import tpu as pltpu
from jax.experimental.pallas import tpu_sc as plsc

B = 16384
D = 32
NC = 2
NS = 16
L = 16
NW = NC * NS
BPW = B // NW
ICH = 128
NCHUNK = BPW // ICH
NBLK = BPW // L

_mesh = plsc.VectorSubcoreMesh(core_axis_name="c", subcore_axis_name="s")


@functools.partial(
    pl.kernel,
    out_type=(
        jax.ShapeDtypeStruct((B,), jnp.float32),
        jax.ShapeDtypeStruct((B,), jnp.float32),
    ),
    mesh=_mesh,
    compiler_params=pltpu.CompilerParams(
        needs_layout_passes=False, use_tc_tiling_on_sc=False),
    scratch_types=[
        pltpu.VMEM((NCHUNK, ICH), jnp.int32),
        pltpu.VMEM((NCHUNK, ICH), jnp.int32),
        pltpu.VMEM((NCHUNK, ICH), jnp.int32),
        pltpu.VMEM((BPW, D), jnp.float32),
        pltpu.VMEM((BPW, D), jnp.float32),
        pltpu.VMEM((BPW, D), jnp.float32),
        pltpu.VMEM((BPW,), jnp.float32),
        pltpu.VMEM((BPW,), jnp.float32),
        pltpu.SemaphoreType.DMA,
    ],
)
def _bpr_sc(uidx_hbm, pidx_hbm, nidx_hbm, utab_hbm, itab_hbm,
            pos_hbm, neg_hbm,
            uidx_v, pidx_v, nidx_v, u_v, p_v, n_v, po_v, no_v, sem):
    wid = lax.axis_index("s") * NC + lax.axis_index("c")
    base = wid * BPW

    pltpu.sync_copy(uidx_hbm.at[wid], uidx_v)
    pltpu.sync_copy(pidx_hbm.at[wid], pidx_v)
    pltpu.sync_copy(nidx_hbm.at[wid], nidx_v)

    copies = []
    for j in range(NCHUNK):
        dst = pl.ds(j * ICH, ICH)
        copies.append(pltpu.async_copy(utab_hbm.at[uidx_v.at[j]], u_v.at[dst], sem))
        copies.append(pltpu.async_copy(itab_hbm.at[pidx_v.at[j]], p_v.at[dst], sem))
        copies.append(pltpu.async_copy(itab_hbm.at[nidx_v.at[j]], n_v.at[dst], sem))
    for c in copies:
        c.wait()

    lanes = lax.iota(jnp.int32, L)

    def blk(b, carry):
        rows = b * L + lanes
        accp = jnp.zeros((L,), jnp.float32)
        accn = jnp.zeros((L,), jnp.float32)
        for d in range(D):
            dd = jnp.full((L,), d, jnp.int32)
            uu = plsc.load_gather(u_v, [rows, dd])
            pp = plsc.load_gather(p_v, [rows, dd])
            nn = plsc.load_gather(n_v, [rows, dd])
            accp = accp + uu * pp
            accn = accn + uu * nn
        po_v[pl.ds(b * L, L)] = accp
        no_v[pl.ds(b * L, L)] = accn
        return carry

    lax.fori_loop(0, NBLK, blk, 0)

    pltpu.sync_copy(po_v, pos_hbm.at[pl.ds(base, BPW)])
    pltpu.sync_copy(no_v, neg_hbm.at[pl.ds(base, BPW)])


def kernel(user_inputs, pos_inputs, neg_inputs, user_table, item_table):
    u = user_inputs.reshape(NW, NCHUNK, ICH)
    p = pos_inputs.reshape(NW, NCHUNK, ICH)
    n = neg_inputs.reshape(NW, NCHUNK, ICH)
    pos, neg = _bpr_sc(u, p, n, user_table, item_table)
    return pos.reshape(B, 1), neg.reshape(B, 1)


# speedup vs baseline: 2.1640x; 2.1640x over previous
import functools

import jax
import jax.numpy as jnp
from jax import lax
from jax.experimental import pallas as pl
from jax.experimental.pallas import tpu as pltpu
from jax.experimental.pallas import tpu_sc as plsc

B = 16384
D = 32
NC = 2
NS = 16
L = 16
NW = NC * NS
BPW = B // NW

V = 1000000
CHW = 1024
VMAIN = 999936
NCHUNKS = VMAIN // CHW + 1
LASTW = VMAIN - (NCHUNKS - 1) * CHW
CPT = (NCHUNKS + NW - 1) // NW
HCAP = 4096
ISTAGE = 4096

_mesh = plsc.VectorSubcoreMesh(
    core_axis_name="c", subcore_axis_name="s", num_cores=NC, num_subcores=NS)


def _make_k1(n):
    nrounds = (n + HCAP - 1) // HCAP

    @functools.partial(
        pl.kernel,
        out_type=jax.ShapeDtypeStruct((n * D,), jnp.float32),
        mesh=_mesh,
        compiler_params=pltpu.CompilerParams(
            use_tc_tiling_on_sc=True, needs_layout_passes=False),
        scratch_types=[
            pltpu.VMEM((ISTAGE,), jnp.int32),
            pltpu.VMEM((HCAP + L,), jnp.int32),
            pltpu.VMEM((HCAP + L,), jnp.int32),
            pltpu.VMEM((HCAP + L,), jnp.int32),
            pltpu.VMEM((HCAP + L,), jnp.int32),
            pltpu.VMEM((D, CHW), jnp.float32),
            pltpu.VMEM((16, 128), jnp.float32),
            pltpu.VMEM((D,), jnp.float32),
            pltpu.SemaphoreType.DMA,
        ],
    )
    def k1(idx_hbm, tabT, tail_hbm, emb_out,
           istage_v, hidx_v, hj_v, cidx_v, cj_v, chunk_v, tail_v, row_v, sem):
        wid = lax.axis_index("s") * NC + lax.axis_index("c")
        lanes = lax.iota(jnp.int32, L)
        d16a = lax.iota(jnp.int32, L)
        d16b = d16a + L

        pltpu.sync_copy(tail_hbm, tail_v)

        def extract(cc, use_tail_sel):
            if use_tail_sel:
                sel = cc < LASTW
                ccv = jnp.full((L,), jnp.minimum(cc, LASTW - 1), jnp.int32)
                fa = (cc - LASTW) * D + d16a
                fb = (cc - LASTW) * D + d16b
                la = plsc.load_gather(chunk_v, [d16a, ccv])
                lb = plsc.load_gather(chunk_v, [d16b, ccv])
                zero = jnp.zeros((L,), jnp.int32)
                ta = plsc.load_gather(
                    tail_v, [jnp.maximum(fa, zero) >> 7, jnp.maximum(fa, zero) & 127])
                tb = plsc.load_gather(
                    tail_v, [jnp.maximum(fb, zero) >> 7, jnp.maximum(fb, zero) & 127])
                selv = jnp.full((L,), sel, jnp.bool_)
                return jnp.where(selv, la, ta), jnp.where(selv, lb, tb)
            ccv = jnp.full((L,), cc, jnp.int32)
            return (plsc.load_gather(chunk_v, [d16a, ccv]),
                    plsc.load_gather(chunk_v, [d16b, ccv]))

        def round_body(r, carry_in):
            total_done, known_rank = carry_in
            lo = r * HCAP

            def stage_loop(s, carry):
                pltpu.sync_copy(idx_hbm.at[pl.ds(s * ISTAGE, ISTAGE)], istage_v)

                def filt(b, carry2):
                    cnt2, rank2 = carry2
                    v = istage_v[pl.ds(b * L, L)]
                    j = (s * ISTAGE + b * L) + lanes
                    m = ((v >> 10) & (NW - 1)) == wid
                    pref = plsc.cumsum(m.astype(jnp.int32))
                    rk = rank2 + pref - 1
                    mw = m & (rk >= lo) & (rk < lo + HCAP)
                    nh = plsc.all_reduce_population_count(mw)[0]
                    plsc.store_compressed(hidx_v.at[pl.ds(cnt2, L)], v, mask=mw)
                    plsc.store_compressed(hj_v.at[pl.ds(cnt2, L)], j, mask=mw)
                    nm = plsc.all_reduce_population_count(m)[0]
                    return cnt2 + nh, rank2 + nm

                return lax.fori_loop(0, ISTAGE // L, filt, carry)

            def run_filter(_):
                return lax.fori_loop(0, n // ISTAGE, stage_loop,
                                     (jnp.int32(0), jnp.int32(0)))

            def skip_filter(_):
                return jnp.int32(0), known_rank

            active = (r == 0) | (lo < known_rank)
            nhits, nrank = lax.cond(active, run_filter, skip_filter, 0)

            @pl.when(nhits > 0)
            def _():
                def chunk_slot(cs, car):
                    c = cs * NW + wid

                    @pl.when(c < NCHUNKS)
                    def _():
                        cbase = c * CHW

                        def cfilt(b, cnt2):
                            v = hidx_v[pl.ds(b * L, L)]
                            j = hj_v[pl.ds(b * L, L)]
                            inlist = (b * L + lanes) < nhits
                            m = inlist & ((v >> 10) == c)
                            nh = plsc.all_reduce_population_count(m)[0]
                            plsc.store_compressed(
                                cidx_v.at[pl.ds(cnt2, L)], v, mask=m)
                            plsc.store_compressed(
                                cj_v.at[pl.ds(cnt2, L)], j, mask=m)
                            return cnt2 + nh

                        nb = (nhits + L - 1) // L
                        nc_ = lax.fori_loop(0, nb, cfilt, jnp.int32(0))

                        @pl.when(nc_ > 0)
                        def _():
                            @pl.when(c < NCHUNKS - 1)
                            def _():
                                pltpu.sync_copy(
                                    tabT.at[:, pl.ds(cbase, CHW)], chunk_v)

                                def hit_loop(h, u):
                                    cc = cidx_v[pl.ds(h, L)][0] - cbase
                                    j = cj_v[pl.ds(h, L)][0]
                                    a, b_ = extract(cc, False)
                                    row_v[pl.ds(0, L)] = a
                                    row_v[pl.ds(L, L)] = b_
                                    pltpu.sync_copy(
                                        row_v, emb_out.at[pl.ds(j * D, D)])
                                    return u

                                lax.fori_loop(0, nc_, hit_loop, 0)

                            @pl.when(c == NCHUNKS - 1)
                            def _():
                                pltpu.sync_copy(
                                    tabT.at[:, pl.ds((NCHUNKS - 1) * CHW, LASTW)],
                                    chunk_v.at[:, pl.ds(0, LASTW)])

                                def hit_loop(h, u):
                                    cc = cidx_v[pl.ds(h, L)][0] - cbase
                                    j = cj_v[pl.ds(h, L)][0]
                                    a, b_ = extract(cc, True)
                                    row_v[pl.ds(0, L)] = a
                                    row_v[pl.ds(L, L)] = b_
                                    pltpu.sync_copy(
                                        row_v, emb_out.at[pl.ds(j * D, D)])
                                    return u

                                lax.fori_loop(0, nc_, hit_loop, 0)

                    return car

                lax.fori_loop(0, CPT, chunk_slot, 0)

            return total_done + nhits, nrank

        lax.fori_loop(0, nrounds, round_body, (jnp.int32(0), jnp.int32(0)))

    return k1


_k1_user = _make_k1(B)
_k1_item = _make_k1(2 * B)


@functools.partial(
    pl.kernel,
    out_type=(
        jax.ShapeDtypeStruct((B,), jnp.float32),
        jax.ShapeDtypeStruct((B,), jnp.float32),
    ),
    mesh=_mesh,
    compiler_params=pltpu.CompilerParams(
        needs_layout_passes=False, use_tc_tiling_on_sc=False),
    scratch_types=[
        pltpu.VMEM((BPW, D), jnp.float32),
        pltpu.VMEM((BPW, D), jnp.float32),
        pltpu.VMEM((BPW, D), jnp.float32),
        pltpu.VMEM((BPW,), jnp.float32),
        pltpu.VMEM((BPW,), jnp.float32),
    ],
)
def _k2(emb_u_hbm, emb_pn_hbm, pos_hbm, neg_hbm,
        u_v, p_v, n_v, po_v, no_v):
    wid = lax.axis_index("s") * NC + lax.axis_index("c")
    base = wid * BPW

    pltpu.sync_copy(emb_u_hbm.at[pl.ds(base, BPW)], u_v)
    pltpu.sync_copy(emb_pn_hbm.at[pl.ds(base, BPW)], p_v)
    pltpu.sync_copy(emb_pn_hbm.at[pl.ds(B + base, BPW)], n_v)

    lanes = lax.iota(jnp.int32, L)

    def blk(b, carry):
        rows = b * L + lanes
        accp = jnp.zeros((L,), jnp.float32)
        accn = jnp.zeros((L,), jnp.float32)
        for d in range(D):
            dd = jnp.full((L,), d, jnp.int32)
            uu = plsc.load_gather(u_v, [rows, dd])
            pp = plsc.load_gather(p_v, [rows, dd])
            nn = plsc.load_gather(n_v, [rows, dd])
            accp = accp + uu * pp
            accn = accn + uu * nn
        po_v[pl.ds(b * L, L)] = accp
        no_v[pl.ds(b * L, L)] = accn
        return carry

    lax.fori_loop(0, BPW // L, blk, 0)

    pltpu.sync_copy(po_v, pos_hbm.at[pl.ds(base, BPW)])
    pltpu.sync_copy(no_v, neg_hbm.at[pl.ds(base, BPW)])


def kernel(user_inputs, pos_inputs, neg_inputs, user_table, item_table):
    uidx = user_inputs.reshape(B)
    pnidx = jnp.concatenate([pos_inputs.reshape(B), neg_inputs.reshape(B)])
    utail = user_table[VMAIN:].reshape(16, 128)
    itail = item_table[VMAIN:].reshape(16, 128)
    emb_u = _k1_user(uidx, user_table.T, utail)
    emb_pn = _k1_item(pnidx, item_table.T, itail)
    pos, neg = _k2(emb_u.reshape(B, D), emb_pn.reshape(2 * B, D))
    return pos.reshape(B, 1), neg.reshape(B, 1)


# speedup vs baseline: 3.0953x; 1.4304x over previous
import functools

import jax
import jax.numpy as jnp
from jax import lax
from jax.experimental import pallas as pl
from jax.experimental.pallas import tpu as pltpu
from jax.experimental.pallas import tpu_sc as plsc

B = 16384
D = 32
NC = 2
NS = 16
L = 16
NW = NC * NS
BPW = B // NW

V = 1000000
CHW = 1024
VMAIN = 999936
NCHUNKS = VMAIN // CHW + 1
LASTW = VMAIN - (NCHUNKS - 1) * CHW
CPT = (NCHUNKS + NW - 1) // NW
HCAP = 8192
ISTAGE = 4096

_mesh = plsc.VectorSubcoreMesh(
    core_axis_name="c", subcore_axis_name="s", num_cores=NC, num_subcores=NS)


def _make_k1(n):
    nrounds = (n + HCAP - 1) // HCAP

    @functools.partial(
        pl.kernel,
        out_type=jax.ShapeDtypeStruct((n * D,), jnp.float32),
        mesh=_mesh,
        compiler_params=pltpu.CompilerParams(
            use_tc_tiling_on_sc=True, needs_layout_passes=False),
        scratch_types=[
            pltpu.VMEM((ISTAGE,), jnp.int32),
            pltpu.VMEM((HCAP + L,), jnp.int32),
            pltpu.VMEM((HCAP + L,), jnp.int32),
            pltpu.VMEM((HCAP + L,), jnp.int32),
            pltpu.VMEM((HCAP + L,), jnp.int32),
            pltpu.VMEM((HCAP + L,), jnp.int32),
            pltpu.VMEM((D, CHW), jnp.float32),
            pltpu.VMEM((D, CHW), jnp.float32),
            pltpu.VMEM((16, 128), jnp.float32),
            pltpu.VMEM((D,), jnp.float32),
            pltpu.SemaphoreType.DMA,
        ],
    )
    def k1(idx_hbm, tabT, tail_hbm, emb_out,
           istage_v, hidx_v, hj_v, hch_v, cidx_v, cj_v,
           buf0_v, buf1_v, tail_v, row_v, sem):
        wid = lax.axis_index("s") * NC + lax.axis_index("c")
        lanes = lax.iota(jnp.int32, L)
        d16a = lax.iota(jnp.int32, L)
        d16b = d16a + L

        pltpu.sync_copy(tail_hbm, tail_v)

        def start_chunk(cs, buf):
            c = cs * NW + wid

            @pl.when(c < NCHUNKS - 1)
            def _():
                pltpu.async_copy(tabT.at[:, pl.ds(c * CHW, CHW)], buf, sem)

            @pl.when(c == NCHUNKS - 1)
            def _():
                pltpu.async_copy(
                    tabT.at[:, pl.ds((NCHUNKS - 1) * CHW, LASTW)],
                    buf.at[:, pl.ds(0, LASTW)], sem)

        def wait_chunk(cs, buf):
            c = cs * NW + wid

            @pl.when(c < NCHUNKS - 1)
            def _():
                pltpu.make_async_copy(
                    tabT.at[:, pl.ds(0, CHW)], buf, sem).wait()

            @pl.when(c == NCHUNKS - 1)
            def _():
                pltpu.make_async_copy(
                    tabT.at[:, pl.ds(0, LASTW)],
                    buf.at[:, pl.ds(0, LASTW)], sem).wait()

        def extract(buf, cc, tailable):
            if tailable:
                sel = cc < LASTW
                ccv = jnp.full((L,), jnp.minimum(cc, LASTW - 1), jnp.int32)
                fa = (cc - LASTW) * D + d16a
                fb = (cc - LASTW) * D + d16b
                la = plsc.load_gather(buf, [d16a, ccv])
                lb = plsc.load_gather(buf, [d16b, ccv])
                zero = jnp.zeros((L,), jnp.int32)
                ta = plsc.load_gather(
                    tail_v, [jnp.maximum(fa, zero) >> 7, jnp.maximum(fa, zero) & 127])
                tb = plsc.load_gather(
                    tail_v, [jnp.maximum(fb, zero) >> 7, jnp.maximum(fb, zero) & 127])
                selv = jnp.full((L,), sel, jnp.bool_)
                return jnp.where(selv, la, ta), jnp.where(selv, lb, tb)
            ccv = jnp.full((L,), cc, jnp.int32)
            return (plsc.load_gather(buf, [d16a, ccv]),
                    plsc.load_gather(buf, [d16b, ccv]))

        def do_chunk(cs, buf, nhits):
            c = cs * NW + wid
            cbase = c * CHW

            def cfilt(b, cnt2):
                ch = hch_v[pl.ds(b * L, L)]
                v = hidx_v[pl.ds(b * L, L)]
                j = hj_v[pl.ds(b * L, L)]
                inlist = (b * L + lanes) < nhits
                m = inlist & (ch == c)
                nh = plsc.all_reduce_population_count(m)[0]
                plsc.store_compressed(cidx_v.at[pl.ds(cnt2, L)], v, mask=m)
                plsc.store_compressed(cj_v.at[pl.ds(cnt2, L)], j, mask=m)
                return cnt2 + nh

            nb = (nhits + L - 1) // L
            nc_ = lax.fori_loop(0, nb, cfilt, jnp.int32(0))

            def make_hit_loop(tailable):
                def hit_loop(h, u):
                    cc = cidx_v[pl.ds(h, L)][0] - cbase
                    j = cj_v[pl.ds(h, L)][0]
                    a, b_ = extract(buf, cc, tailable)
                    row_v[pl.ds(0, L)] = a
                    row_v[pl.ds(L, L)] = b_
                    pltpu.sync_copy(row_v, emb_out.at[pl.ds(j * D, D)])
                    return u
                return hit_loop

            @pl.when(c < NCHUNKS - 1)
            def _():
                lax.fori_loop(0, nc_, make_hit_loop(False), 0)

            @pl.when(c == NCHUNKS - 1)
            def _():
                lax.fori_loop(0, nc_, make_hit_loop(True), 0)

        def scan_chunks(nhits):
            start_chunk(0, buf0_v)

            def pair(t, car):
                a = 2 * t
                b = 2 * t + 1

                @pl.when((b * NW + wid) < NCHUNKS)
                def _():
                    start_chunk(b, buf1_v)

                @pl.when((a * NW + wid) < NCHUNKS)
                def _():
                    wait_chunk(a, buf0_v)
                    do_chunk(a, buf0_v, nhits)

                @pl.when(((a + 2) * NW + wid) < NCHUNKS)
                def _():
                    start_chunk(a + 2, buf0_v)

                @pl.when((b * NW + wid) < NCHUNKS)
                def _():
                    wait_chunk(b, buf1_v)
                    do_chunk(b, buf1_v, nhits)

                return car

            lax.fori_loop(0, (CPT + 2) // 2, pair, 0)

        def stage_fast(s, cnt):
            pltpu.sync_copy(idx_hbm.at[pl.ds(s * ISTAGE, ISTAGE)], istage_v)

            def filt(b, cnt2):
                v = istage_v[pl.ds(b * L, L)]
                j = (s * ISTAGE + b * L) + lanes
                m = ((v >> 10) & (NW - 1)) == wid
                nh = plsc.all_reduce_population_count(m)[0]
                cur = jnp.minimum(cnt2, HCAP)
                plsc.store_compressed(hidx_v.at[pl.ds(cur, L)], v, mask=m)
                plsc.store_compressed(hj_v.at[pl.ds(cur, L)], j, mask=m)
                plsc.store_compressed(hch_v.at[pl.ds(cur, L)], v >> 10, mask=m)
                return cnt2 + nh

            return lax.fori_loop(0, ISTAGE // L, filt, cnt)

        nrank = lax.fori_loop(0, n // ISTAGE, stage_fast, jnp.int32(0))
        scan_chunks(jnp.minimum(nrank, HCAP))

        if nrounds > 1:
            def round_body(r, car):
                lo = r * HCAP

                def stage_win(s, carry):
                    pltpu.sync_copy(
                        idx_hbm.at[pl.ds(s * ISTAGE, ISTAGE)], istage_v)

                    def filt(b, carry2):
                        cnt2, rank2 = carry2
                        v = istage_v[pl.ds(b * L, L)]
                        j = (s * ISTAGE + b * L) + lanes
                        m = ((v >> 10) & (NW - 1)) == wid
                        pref = plsc.cumsum(m.astype(jnp.int32))
                        rk = rank2 + pref - 1
                        mw = m & (rk >= lo) & (rk < lo + HCAP)
                        nh = plsc.all_reduce_population_count(mw)[0]
                        plsc.store_compressed(
                            hidx_v.at[pl.ds(cnt2, L)], v, mask=mw)
                        plsc.store_compressed(
                            hj_v.at[pl.ds(cnt2, L)], j, mask=mw)
                        plsc.store_compressed(
                            hch_v.at[pl.ds(cnt2, L)], v >> 10, mask=mw)
                        nm = plsc.all_reduce_population_count(m)[0]
                        return cnt2 + nh, rank2 + nm

                    return lax.fori_loop(0, ISTAGE // L, filt, carry)

                @pl.when(lo < nrank)
                def _():
                    nhits, _unused = lax.fori_loop(
                        0, n // ISTAGE, stage_win, (jnp.int32(0), jnp.int32(0)))
                    scan_chunks(nhits)

                return car

            lax.fori_loop(1, nrounds, round_body, 0)

    return k1


_k1_user = _make_k1(B)
_k1_item = _make_k1(2 * B)


@functools.partial(
    pl.kernel,
    out_type=(
        jax.ShapeDtypeStruct((B,), jnp.float32),
        jax.ShapeDtypeStruct((B,), jnp.float32),
    ),
    mesh=_mesh,
    compiler_params=pltpu.CompilerParams(
        needs_layout_passes=False, use_tc_tiling_on_sc=False),
    scratch_types=[
        pltpu.VMEM((BPW, D), jnp.float32),
        pltpu.VMEM((BPW, D), jnp.float32),
        pltpu.VMEM((BPW, D), jnp.float32),
        pltpu.VMEM((BPW,), jnp.float32),
        pltpu.VMEM((BPW,), jnp.float32),
    ],
)
def _k2(emb_u_hbm, emb_pn_hbm, pos_hbm, neg_hbm,
        u_v, p_v, n_v, po_v, no_v):
    wid = lax.axis_index("s") * NC + lax.axis_index("c")
    base = wid * BPW

    pltpu.sync_copy(emb_u_hbm.at[pl.ds(base, BPW)], u_v)
    pltpu.sync_copy(emb_pn_hbm.at[pl.ds(base, BPW)], p_v)
    pltpu.sync_copy(emb_pn_hbm.at[pl.ds(B + base, BPW)], n_v)

    lanes = lax.iota(jnp.int32, L)

    def blk(b, carry):
        rows = b * L + lanes
        accp = jnp.zeros((L,), jnp.float32)
        accn = jnp.zeros((L,), jnp.float32)
        for d in range(D):
            dd = jnp.full((L,), d, jnp.int32)
            uu = plsc.load_gather(u_v, [rows, dd])
            pp = plsc.load_gather(p_v, [rows, dd])
            nn = plsc.load_gather(n_v, [rows, dd])
            accp = accp + uu * pp
            accn = accn + uu * nn
        po_v[pl.ds(b * L, L)] = accp
        no_v[pl.ds(b * L, L)] = accn
        return carry

    lax.fori_loop(0, BPW // L, blk, 0)

    pltpu.sync_copy(po_v, pos_hbm.at[pl.ds(base, BPW)])
    pltpu.sync_copy(no_v, neg_hbm.at[pl.ds(base, BPW)])


def kernel(user_inputs, pos_inputs, neg_inputs, user_table, item_table):
    uidx = user_inputs.reshape(B)
    pnidx = jnp.concatenate([pos_inputs.reshape(B), neg_inputs.reshape(B)])
    utail = user_table[VMAIN:].reshape(16, 128)
    itail = item_table[VMAIN:].reshape(16, 128)
    emb_u = _k1_user(uidx, user_table.T, utail)
    emb_pn = _k1_item(pnidx, item_table.T, itail)
    pos, neg = _k2(emb_u.reshape(B, D), emb_pn.reshape(2 * B, D))
    return pos.reshape(B, 1), neg.reshape(B, 1)


# speedup vs baseline: 4.1849x; 1.3520x over previous
import functools

import jax
import jax.numpy as jnp
from jax import lax
from jax.experimental import pallas as pl
from jax.experimental.pallas import tpu as pltpu
from jax.experimental.pallas import tpu_sc as plsc

B = 16384
D = 32
NC = 2
NS = 16
L = 16
NW = NC * NS
BPW = B // NW

V = 1000000
CHW = 1024
VMAIN = 999936
NCHUNKS = VMAIN // CHW + 1
LASTW = VMAIN - (NCHUNKS - 1) * CHW
CPT = (NCHUNKS + NW - 1) // NW
HCAP = 8192
ISTAGE = 4096

_mesh = plsc.VectorSubcoreMesh(
    core_axis_name="c", subcore_axis_name="s", num_cores=NC, num_subcores=NS)


def _make_k1(n):
    nrounds = (n + HCAP - 1) // HCAP

    @functools.partial(
        pl.kernel,
        out_type=jax.ShapeDtypeStruct((n * D,), jnp.float32),
        mesh=_mesh,
        compiler_params=pltpu.CompilerParams(
            use_tc_tiling_on_sc=True, needs_layout_passes=False),
        scratch_types=[
            pltpu.VMEM((ISTAGE,), jnp.int32),
            pltpu.VMEM((HCAP + L,), jnp.int32),
            pltpu.VMEM((HCAP + L,), jnp.int32),
            pltpu.VMEM((HCAP + L,), jnp.int32),
            pltpu.VMEM((HCAP + L,), jnp.int32),
            pltpu.VMEM((HCAP + L,), jnp.int32),
            pltpu.VMEM((D, CHW), jnp.float32),
            pltpu.VMEM((D, CHW), jnp.float32),
            pltpu.VMEM((16, 128), jnp.float32),
            pltpu.VMEM((L, D), jnp.float32),
            pltpu.SemaphoreType.DMA,
            pltpu.SemaphoreType.DMA,
        ],
    )
    def k1(idx_hbm, tabT, tail_hbm, emb_out,
           istage_v, hidx_v, hj_v, hch_v, cidx_v, cj_v,
           buf0_v, buf1_v, tail_v, rowbank_v, sem, wsem):
        wid = lax.axis_index("s") * NC + lax.axis_index("c")
        lanes = lax.iota(jnp.int32, L)
        d16a = lax.iota(jnp.int32, L)
        d16b = d16a + L

        pltpu.sync_copy(tail_hbm, tail_v)

        def start_chunk(cs, buf):
            c = cs * NW + wid

            @pl.when(c < NCHUNKS - 1)
            def _():
                pltpu.async_copy(tabT.at[:, pl.ds(c * CHW, CHW)], buf, sem)

            @pl.when(c == NCHUNKS - 1)
            def _():
                pltpu.async_copy(
                    tabT.at[:, pl.ds((NCHUNKS - 1) * CHW, LASTW)],
                    buf.at[:, pl.ds(0, LASTW)], sem)

        def wait_chunk(cs, buf):
            c = cs * NW + wid

            @pl.when(c < NCHUNKS - 1)
            def _():
                pltpu.make_async_copy(
                    tabT.at[:, pl.ds(0, CHW)], buf, sem).wait()

            @pl.when(c == NCHUNKS - 1)
            def _():
                pltpu.make_async_copy(
                    tabT.at[:, pl.ds(0, LASTW)],
                    buf.at[:, pl.ds(0, LASTW)], sem).wait()

        def extract(buf, cc, tailable):
            if tailable:
                sel = cc < LASTW
                ccv = jnp.full((L,), jnp.minimum(cc, LASTW - 1), jnp.int32)
                fa = (cc - LASTW) * D + d16a
                fb = (cc - LASTW) * D + d16b
                la = plsc.load_gather(buf, [d16a, ccv])
                lb = plsc.load_gather(buf, [d16b, ccv])
                zero = jnp.zeros((L,), jnp.int32)
                ta = plsc.load_gather(
                    tail_v, [jnp.maximum(fa, zero) >> 7, jnp.maximum(fa, zero) & 127])
                tb = plsc.load_gather(
                    tail_v, [jnp.maximum(fb, zero) >> 7, jnp.maximum(fb, zero) & 127])
                selv = jnp.full((L,), sel, jnp.bool_)
                return jnp.where(selv, la, ta), jnp.where(selv, lb, tb)
            ccv = jnp.full((L,), cc, jnp.int32)
            return (plsc.load_gather(buf, [d16a, ccv]),
                    plsc.load_gather(buf, [d16b, ccv]))

        def row_drain_one():
            pltpu.make_async_copy(
                emb_out.at[pl.ds(0, D)], rowbank_v.at[0], wsem).wait()

        def cfilt_chunk(cs, nhits):
            c = cs * NW + wid

            def cfilt(b, cnt2):
                ch = hch_v[pl.ds(b * L, L)]
                v = hidx_v[pl.ds(b * L, L)]
                j = hj_v[pl.ds(b * L, L)]
                inlist = (b * L + lanes) < nhits
                m = inlist & (ch == c)
                nh = plsc.all_reduce_population_count(m)[0]
                plsc.store_compressed(cidx_v.at[pl.ds(cnt2, L)], v, mask=m)
                plsc.store_compressed(cj_v.at[pl.ds(cnt2, L)], j, mask=m)
                return cnt2 + nh

            nb = (nhits + L - 1) // L
            return lax.fori_loop(0, nb, cfilt, jnp.int32(0))

        def extract_chunk(cs, buf, nc_):
            c = cs * NW + wid
            cbase = c * CHW

            def make_hit_loop(tailable):
                def hit_loop(h, u):
                    cc = cidx_v[pl.ds(h, L)][0] - cbase
                    j = cj_v[pl.ds(h, L)][0]
                    slot = h & (L - 1)
                    a, b_ = extract(buf, cc, tailable)
                    rowbank_v[slot, pl.ds(0, L)] = a
                    rowbank_v[slot, pl.ds(L, L)] = b_

                    @pl.when(h >= L)
                    def _():
                        row_drain_one()

                    pltpu.async_copy(
                        rowbank_v.at[slot], emb_out.at[pl.ds(j * D, D)], wsem)
                    return u
                return hit_loop

            @pl.when(c < NCHUNKS - 1)
            def _():
                lax.fori_loop(0, nc_, make_hit_loop(False), 0)

            @pl.when(c == NCHUNKS - 1)
            def _():
                lax.fori_loop(0, nc_, make_hit_loop(True), 0)

            def drain(i, u):
                row_drain_one()
                return u

            lax.fori_loop(0, jnp.minimum(nc_, L), drain, 0)

        def do_chunk(cs, buf, nhits):
            nc_ = cfilt_chunk(cs, nhits)
            wait_chunk(cs, buf)
            extract_chunk(cs, buf, nc_)

        def scan_chunks(nhits, primed):
            if not primed:
                start_chunk(0, buf0_v)

            def pair(t, car):
                a = 2 * t
                b = 2 * t + 1

                @pl.when((b * NW + wid) < NCHUNKS)
                def _():
                    start_chunk(b, buf1_v)

                @pl.when((a * NW + wid) < NCHUNKS)
                def _():
                    do_chunk(a, buf0_v, nhits)

                @pl.when(((a + 2) * NW + wid) < NCHUNKS)
                def _():
                    start_chunk(a + 2, buf0_v)

                @pl.when((b * NW + wid) < NCHUNKS)
                def _():
                    do_chunk(b, buf1_v, nhits)

                return car

            lax.fori_loop(0, (CPT + 2) // 2, pair, 0)

        start_chunk(0, buf0_v)

        def stage_fast(s, cnt):
            pltpu.sync_copy(idx_hbm.at[pl.ds(s * ISTAGE, ISTAGE)], istage_v)

            def filt(b, cnt2):
                v = istage_v[pl.ds(b * L, L)]
                j = (s * ISTAGE + b * L) + lanes
                m = ((v >> 10) & (NW - 1)) == wid
                nh = plsc.all_reduce_population_count(m)[0]
                cur = jnp.minimum(cnt2, HCAP)
                plsc.store_compressed(hidx_v.at[pl.ds(cur, L)], v, mask=m)
                plsc.store_compressed(hj_v.at[pl.ds(cur, L)], j, mask=m)
                plsc.store_compressed(hch_v.at[pl.ds(cur, L)], v >> 10, mask=m)
                return cnt2 + nh

            return lax.fori_loop(0, ISTAGE // L, filt, cnt)

        nrank = lax.fori_loop(0, n // ISTAGE, stage_fast, jnp.int32(0))
        scan_chunks(jnp.minimum(nrank, HCAP), primed=True)

        if nrounds > 1:
            def round_body(r, car):
                lo = r * HCAP

                def stage_win(s, carry):
                    pltpu.sync_copy(
                        idx_hbm.at[pl.ds(s * ISTAGE, ISTAGE)], istage_v)

                    def filt(b, carry2):
                        cnt2, rank2 = carry2
                        v = istage_v[pl.ds(b * L, L)]
                        j = (s * ISTAGE + b * L) + lanes
                        m = ((v >> 10) & (NW - 1)) == wid
                        pref = plsc.cumsum(m.astype(jnp.int32))
                        rk = rank2 + pref - 1
                        mw = m & (rk >= lo) & (rk < lo + HCAP)
                        nh = plsc.all_reduce_population_count(mw)[0]
                        plsc.store_compressed(
                            hidx_v.at[pl.ds(cnt2, L)], v, mask=mw)
                        plsc.store_compressed(
                            hj_v.at[pl.ds(cnt2, L)], j, mask=mw)
                        plsc.store_compressed(
                            hch_v.at[pl.ds(cnt2, L)], v >> 10, mask=mw)
                        nm = plsc.all_reduce_population_count(m)[0]
                        return cnt2 + nh, rank2 + nm

                    return lax.fori_loop(0, ISTAGE // L, filt, carry)

                @pl.when(lo < nrank)
                def _():
                    nhits, _unused = lax.fori_loop(
                        0, n // ISTAGE, stage_win, (jnp.int32(0), jnp.int32(0)))
                    scan_chunks(nhits, primed=False)

                return car

            lax.fori_loop(1, nrounds, round_body, 0)

    return k1


_k1_user = _make_k1(B)
_k1_item = _make_k1(2 * B)


@functools.partial(
    pl.kernel,
    out_type=(
        jax.ShapeDtypeStruct((B,), jnp.float32),
        jax.ShapeDtypeStruct((B,), jnp.float32),
    ),
    mesh=_mesh,
    compiler_params=pltpu.CompilerParams(
        needs_layout_passes=False, use_tc_tiling_on_sc=False),
    scratch_types=[
        pltpu.VMEM((BPW, D), jnp.float32),
        pltpu.VMEM((BPW, D), jnp.float32),
        pltpu.VMEM((BPW, D), jnp.float32),
        pltpu.VMEM((BPW,), jnp.float32),
        pltpu.VMEM((BPW,), jnp.float32),
    ],
)
def _k2(emb_u_hbm, emb_pn_hbm, pos_hbm, neg_hbm,
        u_v, p_v, n_v, po_v, no_v):
    wid = lax.axis_index("s") * NC + lax.axis_index("c")
    base = wid * BPW

    pltpu.sync_copy(emb_u_hbm.at[pl.ds(base, BPW)], u_v)
    pltpu.sync_copy(emb_pn_hbm.at[pl.ds(base, BPW)], p_v)
    pltpu.sync_copy(emb_pn_hbm.at[pl.ds(B + base, BPW)], n_v)

    lanes = lax.iota(jnp.int32, L)

    def blk(b, carry):
        rows = b * L + lanes
        accp = jnp.zeros((L,), jnp.float32)
        accn = jnp.zeros((L,), jnp.float32)
        for d in range(D):
            dd = jnp.full((L,), d, jnp.int32)
            uu = plsc.load_gather(u_v, [rows, dd])
            pp = plsc.load_gather(p_v, [rows, dd])
            nn = plsc.load_gather(n_v, [rows, dd])
            accp = accp + uu * pp
            accn = accn + uu * nn
        po_v[pl.ds(b * L, L)] = accp
        no_v[pl.ds(b * L, L)] = accn
        return carry

    lax.fori_loop(0, BPW // L, blk, 0)

    pltpu.sync_copy(po_v, pos_hbm.at[pl.ds(base, BPW)])
    pltpu.sync_copy(no_v, neg_hbm.at[pl.ds(base, BPW)])


def kernel(user_inputs, pos_inputs, neg_inputs, user_table, item_table):
    uidx = user_inputs.reshape(B)
    pnidx = jnp.concatenate([pos_inputs.reshape(B), neg_inputs.reshape(B)])
    utail = user_table[VMAIN:].reshape(16, 128)
    itail = item_table[VMAIN:].reshape(16, 128)
    emb_u = _k1_user(uidx, user_table.T, utail)
    emb_pn = _k1_item(pnidx, item_table.T, itail)
    pos, neg = _k2(emb_u.reshape(B, D), emb_pn.reshape(2 * B, D))
    return pos.reshape(B, 1), neg.reshape(B, 1)


# speedup vs baseline: 4.5296x; 1.0824x over previous
import functools

import jax
import jax.numpy as jnp
from jax import lax
from jax.experimental import pallas as pl
from jax.experimental.pallas import tpu as pltpu
from jax.experimental.pallas import tpu_sc as plsc

B = 16384
D = 32
NC = 2
NS = 16
L = 16
NW = NC * NS
BPW = B // NW

V = 1000000
CHW = 1024
VMAIN = 999936
NCHUNKS = VMAIN // CHW + 1
LASTW = VMAIN - (NCHUNKS - 1) * CHW
CPT = (NCHUNKS + NW - 1) // NW
HCAP = 8192
ISTAGE = 4096

_mesh = plsc.VectorSubcoreMesh(
    core_axis_name="c", subcore_axis_name="s", num_cores=NC, num_subcores=NS)


def _make_k1(n):
    nrounds = (n + HCAP - 1) // HCAP

    @functools.partial(
        pl.kernel,
        out_type=jax.ShapeDtypeStruct((n * D,), jnp.float32),
        mesh=_mesh,
        compiler_params=pltpu.CompilerParams(
            use_tc_tiling_on_sc=True, needs_layout_passes=False),
        scratch_types=[
            pltpu.VMEM((ISTAGE,), jnp.int32),
            pltpu.VMEM((HCAP + L,), jnp.int32),
            pltpu.VMEM((HCAP + L,), jnp.int32),
            pltpu.VMEM((HCAP + L,), jnp.int32),
            pltpu.VMEM((HCAP + L,), jnp.int32),
            pltpu.VMEM((HCAP + L,), jnp.int32),
            pltpu.VMEM((D, CHW), jnp.float32),
            pltpu.VMEM((D, CHW), jnp.float32),
            pltpu.VMEM((16, 128), jnp.float32),
            pltpu.VMEM((L, D), jnp.float32),
            pltpu.SemaphoreType.DMA,
            pltpu.SemaphoreType.DMA,
        ],
    )
    def k1(idx_hbm, tabT, tail_hbm, emb_out,
           istage_v, hidx_v, hj_v, hch_v, cidx_v, cj_v,
           buf0_v, buf1_v, tail_v, rowbank_v, sem, wsem):
        wid = lax.axis_index("s") * NC + lax.axis_index("c")
        lanes = lax.iota(jnp.int32, L)
        d16a = lax.iota(jnp.int32, L)
        d16b = d16a + L

        pltpu.sync_copy(tail_hbm, tail_v)

        def start_chunk(cs, buf):
            c = cs * NW + wid

            @pl.when(c < NCHUNKS - 1)
            def _():
                pltpu.async_copy(tabT.at[:, pl.ds(c * CHW, CHW)], buf, sem)

            @pl.when(c == NCHUNKS - 1)
            def _():
                pltpu.async_copy(
                    tabT.at[:, pl.ds((NCHUNKS - 1) * CHW, LASTW)],
                    buf.at[:, pl.ds(0, LASTW)], sem)

        def wait_chunk(cs, buf):
            c = cs * NW + wid

            @pl.when(c < NCHUNKS - 1)
            def _():
                pltpu.make_async_copy(
                    tabT.at[:, pl.ds(0, CHW)], buf, sem).wait()

            @pl.when(c == NCHUNKS - 1)
            def _():
                pltpu.make_async_copy(
                    tabT.at[:, pl.ds(0, LASTW)],
                    buf.at[:, pl.ds(0, LASTW)], sem).wait()

        def extract(buf, cc, tailable):
            if tailable:
                sel = cc < LASTW
                ccv = jnp.full((L,), jnp.minimum(cc, LASTW - 1), jnp.int32)
                fa = (cc - LASTW) * D + d16a
                fb = (cc - LASTW) * D + d16b
                la = plsc.load_gather(buf, [d16a, ccv])
                lb = plsc.load_gather(buf, [d16b, ccv])
                zero = jnp.zeros((L,), jnp.int32)
                ta = plsc.load_gather(
                    tail_v, [jnp.maximum(fa, zero) >> 7, jnp.maximum(fa, zero) & 127])
                tb = plsc.load_gather(
                    tail_v, [jnp.maximum(fb, zero) >> 7, jnp.maximum(fb, zero) & 127])
                selv = jnp.full((L,), sel, jnp.bool_)
                return jnp.where(selv, la, ta), jnp.where(selv, lb, tb)
            ccv = jnp.full((L,), cc, jnp.int32)
            return (plsc.load_gather(buf, [d16a, ccv]),
                    plsc.load_gather(buf, [d16b, ccv]))

        def row_drain_one():
            pltpu.make_async_copy(
                emb_out.at[pl.ds(0, D)], rowbank_v.at[0], wsem).wait()

        def cfilt_chunk(cs, nhits):
            c = cs * NW + wid

            def cfilt(b, cnt2):
                vs, js, ms, nhs = [], [], [], []
                for k in range(2):
                    off = (b * 2 + k) * L
                    ch = hch_v[pl.ds(off, L)]
                    inlist = (off + lanes) < nhits
                    m = inlist & (ch == c)
                    vs.append(hidx_v[pl.ds(off, L)])
                    js.append(hj_v[pl.ds(off, L)])
                    ms.append(m)
                    nhs.append(plsc.all_reduce_population_count(m)[0])
                for k in range(2):
                    plsc.store_compressed(cidx_v.at[pl.ds(cnt2, L)], vs[k], mask=ms[k])
                    plsc.store_compressed(cj_v.at[pl.ds(cnt2, L)], js[k], mask=ms[k])
                    cnt2 = cnt2 + nhs[k]
                return cnt2

            nb = (nhits + 2 * L - 1) // (2 * L)
            return lax.fori_loop(0, nb, cfilt, jnp.int32(0))

        def extract_chunk(cs, buf, nc_):
            c = cs * NW + wid
            cbase = c * CHW

            def make_hit_loop(tailable):
                def hit_loop(h, u):
                    cc = cidx_v[pl.ds(h, L)][0] - cbase
                    j = cj_v[pl.ds(h, L)][0]
                    slot = h & (L - 1)
                    a, b_ = extract(buf, cc, tailable)
                    rowbank_v[slot, pl.ds(0, L)] = a
                    rowbank_v[slot, pl.ds(L, L)] = b_

                    @pl.when(h >= L)
                    def _():
                        row_drain_one()

                    pltpu.async_copy(
                        rowbank_v.at[slot], emb_out.at[pl.ds(j * D, D)], wsem)
                    return u
                return hit_loop

            @pl.when(c < NCHUNKS - 1)
            def _():
                lax.fori_loop(0, nc_, make_hit_loop(False), 0)

            @pl.when(c == NCHUNKS - 1)
            def _():
                lax.fori_loop(0, nc_, make_hit_loop(True), 0)

            def drain(i, u):
                row_drain_one()
                return u

            lax.fori_loop(0, jnp.minimum(nc_, L), drain, 0)

        def do_chunk(cs, buf, nhits):
            nc_ = cfilt_chunk(cs, nhits)
            wait_chunk(cs, buf)
            extract_chunk(cs, buf, nc_)

        def scan_chunks(nhits, primed):
            if not primed:
                start_chunk(0, buf0_v)

            def pair(t, car):
                a = 2 * t
                b = 2 * t + 1

                @pl.when((b * NW + wid) < NCHUNKS)
                def _():
                    start_chunk(b, buf1_v)

                @pl.when((a * NW + wid) < NCHUNKS)
                def _():
                    do_chunk(a, buf0_v, nhits)

                @pl.when(((a + 2) * NW + wid) < NCHUNKS)
                def _():
                    start_chunk(a + 2, buf0_v)

                @pl.when((b * NW + wid) < NCHUNKS)
                def _():
                    do_chunk(b, buf1_v, nhits)

                return car

            lax.fori_loop(0, (CPT + 2) // 2, pair, 0)

        start_chunk(0, buf0_v)

        def stage_fast(s, cnt):
            pltpu.sync_copy(idx_hbm.at[pl.ds(s * ISTAGE, ISTAGE)], istage_v)

            def filt(b, cnt2):
                vs, ms, nhs = [], [], []
                for k in range(4):
                    v = istage_v[pl.ds((b * 4 + k) * L, L)]
                    m = ((v >> 10) & (NW - 1)) == wid
                    vs.append(v)
                    ms.append(m)
                    nhs.append(plsc.all_reduce_population_count(m)[0])
                for k in range(4):
                    j = (s * ISTAGE + (b * 4 + k) * L) + lanes
                    cur = jnp.minimum(cnt2, HCAP)
                    plsc.store_compressed(hidx_v.at[pl.ds(cur, L)], vs[k], mask=ms[k])
                    plsc.store_compressed(hj_v.at[pl.ds(cur, L)], j, mask=ms[k])
                    plsc.store_compressed(hch_v.at[pl.ds(cur, L)], vs[k] >> 10, mask=ms[k])
                    cnt2 = cnt2 + nhs[k]
                return cnt2

            return lax.fori_loop(0, ISTAGE // L // 4, filt, cnt)

        nrank = lax.fori_loop(0, n // ISTAGE, stage_fast, jnp.int32(0))
        scan_chunks(jnp.minimum(nrank, HCAP), primed=True)

        if nrounds > 1:
            def round_body(r, car):
                lo = r * HCAP

                def stage_win(s, carry):
                    pltpu.sync_copy(
                        idx_hbm.at[pl.ds(s * ISTAGE, ISTAGE)], istage_v)

                    def filt(b, carry2):
                        cnt2, rank2 = carry2
                        v = istage_v[pl.ds(b * L, L)]
                        j = (s * ISTAGE + b * L) + lanes
                        m = ((v >> 10) & (NW - 1)) == wid
                        pref = plsc.cumsum(m.astype(jnp.int32))
                        rk = rank2 + pref - 1
                        mw = m & (rk >= lo) & (rk < lo + HCAP)
                        nh = plsc.all_reduce_population_count(mw)[0]
                        plsc.store_compressed(
                            hidx_v.at[pl.ds(cnt2, L)], v, mask=mw)
                        plsc.store_compressed(
                            hj_v.at[pl.ds(cnt2, L)], j, mask=mw)
                        plsc.store_compressed(
                            hch_v.at[pl.ds(cnt2, L)], v >> 10, mask=mw)
                        nm = plsc.all_reduce_population_count(m)[0]
                        return cnt2 + nh, rank2 + nm

                    return lax.fori_loop(0, ISTAGE // L, filt, carry)

                @pl.when(lo < nrank)
                def _():
                    nhits, _unused = lax.fori_loop(
                        0, n // ISTAGE, stage_win, (jnp.int32(0), jnp.int32(0)))
                    scan_chunks(nhits, primed=False)

                return car

            lax.fori_loop(1, nrounds, round_body, 0)

    return k1


_k1_user = _make_k1(B)
_k1_item = _make_k1(2 * B)


@functools.partial(
    pl.kernel,
    out_type=(
        jax.ShapeDtypeStruct((B,), jnp.float32),
        jax.ShapeDtypeStruct((B,), jnp.float32),
    ),
    mesh=_mesh,
    compiler_params=pltpu.CompilerParams(
        needs_layout_passes=False, use_tc_tiling_on_sc=False),
    scratch_types=[
        pltpu.VMEM((BPW, D), jnp.float32),
        pltpu.VMEM((BPW, D), jnp.float32),
        pltpu.VMEM((BPW, D), jnp.float32),
        pltpu.VMEM((BPW,), jnp.float32),
        pltpu.VMEM((BPW,), jnp.float32),
        pltpu.SemaphoreType.DMA,
    ],
)
def _k2(emb_u_hbm, emb_pn_hbm, pos_hbm, neg_hbm,
        u_v, p_v, n_v, po_v, no_v, sem):
    wid = lax.axis_index("s") * NC + lax.axis_index("c")
    base = wid * BPW

    cps = [
        pltpu.async_copy(emb_u_hbm.at[pl.ds(base, BPW)], u_v, sem),
        pltpu.async_copy(emb_pn_hbm.at[pl.ds(base, BPW)], p_v, sem),
        pltpu.async_copy(emb_pn_hbm.at[pl.ds(B + base, BPW)], n_v, sem),
    ]
    for cp in cps:
        cp.wait()

    lanes = lax.iota(jnp.int32, L)

    def blk(b, carry):
        rows = b * L + lanes
        accp = jnp.zeros((L,), jnp.float32)
        accn = jnp.zeros((L,), jnp.float32)
        for d in range(D):
            dd = jnp.full((L,), d, jnp.int32)
            uu = plsc.load_gather(u_v, [rows, dd])
            pp = plsc.load_gather(p_v, [rows, dd])
            nn = plsc.load_gather(n_v, [rows, dd])
            accp = accp + uu * pp
            accn = accn + uu * nn
        po_v[pl.ds(b * L, L)] = accp
        no_v[pl.ds(b * L, L)] = accn
        return carry

    lax.fori_loop(0, BPW // L, blk, 0)

    pltpu.sync_copy(po_v, pos_hbm.at[pl.ds(base, BPW)])
    pltpu.sync_copy(no_v, neg_hbm.at[pl.ds(base, BPW)])


def kernel(user_inputs, pos_inputs, neg_inputs, user_table, item_table):
    uidx = user_inputs.reshape(B)
    pnidx = jnp.concatenate([pos_inputs.reshape(B), neg_inputs.reshape(B)])
    utail = user_table[VMAIN:].reshape(16, 128)
    itail = item_table[VMAIN:].reshape(16, 128)
    emb_u = _k1_user(uidx, user_table.T, utail)
    emb_pn = _k1_item(pnidx, item_table.T, itail)
    pos, neg = _k2(emb_u.reshape(B, D), emb_pn.reshape(2 * B, D))
    return pos.reshape(B, 1), neg.reshape(B, 1)
